# baseline (speedup 1.0000x reference)
import functools

import jax
import jax.numpy as jnp
from jax import lax
from jax.experimental import pallas as pl
from jax.experimental.pallas import tpu as pltpu
from jax.experimental.pallas import tpu_sc as plsc

NC = 2
NS = 16
NW = NC * NS
L = 16


@functools.lru_cache(maxsize=None)
def _build(B, V, F, D):
    BPW = B // NW
    n_groups = BPW // L

    mesh = plsc.VectorSubcoreMesh(core_axis_name="c", subcore_axis_name="s")

    @functools.partial(
        pl.kernel,
        mesh=mesh,
        out_type=jax.ShapeDtypeStruct((B, D), jnp.float32),
        compiler_params=pltpu.CompilerParams(needs_layout_passes=False),
        scratch_types=[
            pltpu.VMEM((BPW,), jnp.int32),
            pltpu.VMEM((BPW,), jnp.int32),
            pltpu.VMEM((BPW,), jnp.int32),
            pltpu.VMEM((L, D), jnp.float32),
            pltpu.VMEM((L, D), jnp.float32),
            pltpu.SemaphoreType.DMA,
            pltpu.SemaphoreType.DMA,
            pltpu.SemaphoreType.DMA,
            pltpu.SemaphoreType.DMA,
        ],
    )
    def sc_kernel(fpi_hbm, fbi_hbm, val_hbm, fpm_hbm, fbt_hbm, out_hbm,
                  fpi_v, fbi_v, val_v, rows_fb, rows_fp,
                  sem_a, sem_b, sem_c, sem_d):
        wid = lax.axis_index("s") * NC + lax.axis_index("c")
        base = wid * BPW
        pltpu.sync_copy(fpi_hbm.at[pl.ds(base, BPW)], fpi_v)
        pltpu.sync_copy(fbi_hbm.at[pl.ds(base, BPW)], fbi_v)
        pltpu.sync_copy(val_hbm.at[pl.ds(base, BPW)], val_v)

        def group(g, carry):
            off = g * L
            val16 = val_v[pl.ds(off, L)]
            fpi16 = fpi_v[pl.ds(off, L)]
            fbi16 = fbi_v[pl.ds(off, L)]
            mask = val16 != 0
            lanes = lax.iota(jnp.int32, L)
            pos16 = base + off + lanes
            any_valid = jnp.max(val16) > 0
            any_invalid = jnp.min(val16) < 1
            fv_s = jnp.min(jnp.where(mask, lanes, L - 1))
            fi_s = jnp.min(jnp.where(mask, L - 1, lanes))
            fv_vec = lax.broadcast_in_dim(fv_s, (L,), ())
            fi_vec = lax.broadcast_in_dim(fi_s, (L,), ())
            fpi_dup = jnp.take_along_axis(fpi16, fv_vec, axis=0,
                                          mode="promise_in_bounds")
            fbi_dup = jnp.take_along_axis(fbi16, fi_vec, axis=0,
                                          mode="promise_in_bounds")
            posv_dup = jnp.take_along_axis(pos16, fv_vec, axis=0,
                                           mode="promise_in_bounds")
            posi_dup = jnp.take_along_axis(pos16, fi_vec, axis=0,
                                           mode="promise_in_bounds")
            fpi_sel = jnp.where(mask, fpi16, fpi_dup)
            fbi_sel = jnp.where(mask, fbi_dup, fbi16)
            pos_fp = jnp.where(mask, pos16, posv_dup)
            pos_fb = jnp.where(mask, posi_dup, pos16)

            cp_fp = pltpu.async_copy(fpm_hbm.at[fpi_sel], rows_fp, sem_b)
            cp_fb = pltpu.async_copy(fbt_hbm.at[fbi_sel], rows_fb, sem_a)

            cp_fp.wait()
            cp_fb.wait()

            @pl.when(any_valid)
            def _scatter_fp():
                pltpu.async_copy(rows_fp, out_hbm.at[pos_fp], sem_c).wait()

            @pl.when(any_invalid)
            def _scatter_fb():
                pltpu.async_copy(rows_fb, out_hbm.at[pos_fb], sem_d).wait()

            return carry

        lax.fori_loop(0, n_groups, group, 0)

    return sc_kernel


def kernel(fp_idx, fb_idx, is_valid, fingerprint_matrix, fallback_table):
    B = fp_idx.shape[0]
    D = fingerprint_matrix.shape[1]
    sc = _build(B, fingerprint_matrix.shape[0], fallback_table.shape[0], D)
    return sc(fp_idx.astype(jnp.int32),
              fb_idx.astype(jnp.int32),
              is_valid.astype(jnp.int32),
              fingerprint_matrix,
              fallback_table)


# speedup vs baseline: 1.1560x; 1.1560x over previous
import functools

import jax
import jax.numpy as jnp
from jax import lax
from jax.experimental import pallas as pl
from jax.experimental.pallas import tpu as pltpu
from jax.experimental.pallas import tpu_sc as plsc

NC = 2
NS = 16
NW = NC * NS
L = 16


@functools.lru_cache(maxsize=None)
def _build(B, V, F, D):
    BPW = B // NW
    n_groups = BPW // L

    mesh = plsc.VectorSubcoreMesh(core_axis_name="c", subcore_axis_name="s")

    @functools.partial(
        pl.kernel,
        mesh=mesh,
        out_type=jax.ShapeDtypeStruct((B, D), jnp.float32),
        compiler_params=pltpu.CompilerParams(needs_layout_passes=False),
        scratch_types=[
            pltpu.VMEM((BPW,), jnp.int32),
            pltpu.VMEM((BPW,), jnp.int32),
            pltpu.VMEM((BPW,), jnp.int32),
            pltpu.VMEM((2, L, D), jnp.float32),
            pltpu.VMEM((2, L, D), jnp.float32),
            pltpu.SemaphoreType.DMA,
            pltpu.SemaphoreType.DMA,
        ],
    )
    def sc_kernel(fpi_hbm, fbi_hbm, val_hbm, fpm_hbm, fbt_hbm, out_hbm,
                  fpi_v, fbi_v, val_v, rows_fp, rows_fb,
                  sem_g, sem_s):
        wid = lax.axis_index("s") * NC + lax.axis_index("c")
        base = wid * BPW
        pltpu.sync_copy(fpi_hbm.at[pl.ds(base, BPW)], fpi_v)
        pltpu.sync_copy(fbi_hbm.at[pl.ds(base, BPW)], fbi_v)
        pltpu.sync_copy(val_hbm.at[pl.ds(base, BPW)], val_v)

        lanes = lax.iota(jnp.int32, L)

        def params(g):
            off = g * L
            val16 = val_v[pl.ds(off, L)]
            fpi16 = fpi_v[pl.ds(off, L)]
            fbi16 = fbi_v[pl.ds(off, L)]
            mask = val16 != 0
            pos16 = base + off + lanes
            av = jnp.max(val16)
            ai = 1 - jnp.min(val16)
            fv = lax.broadcast_in_dim(jnp.min(jnp.where(mask, lanes, L - 1)),
                                      (L,), ())
            fi = lax.broadcast_in_dim(jnp.min(jnp.where(mask, L - 1, lanes)),
                                      (L,), ())
            take = functools.partial(jnp.take_along_axis, axis=0,
                                     mode="promise_in_bounds")
            fpi_sel = jnp.where(mask, fpi16, take(fpi16, fv))
            fbi_sel = jnp.where(mask, take(fbi16, fi), fbi16)
            pos_fp = jnp.where(mask, pos16, take(pos16, fv))
            pos_fb = jnp.where(mask, take(pos16, fi), pos16)
            return fpi_sel, fbi_sel, pos_fp, pos_fb, av, ai

        def gathers(fpi_sel, fbi_sel, b):
            pltpu.async_copy(fpm_hbm.at[fpi_sel], rows_fp.at[b], sem_g)
            pltpu.async_copy(fbt_hbm.at[fbi_sel], rows_fb.at[b], sem_g)

        fpi0, fbi0, pos_fp0, pos_fb0, av0, ai0 = params(0)
        gathers(fpi0, fbi0, 0)

        def step(g, carry):
            pos_fp_g, pos_fb_g, av_g, ai_g, av_p, ai_p = carry
            b = lax.rem(g, 2)
            bn = 1 - b

            @pl.when(av_p > 0)
            def _():
                pltpu.make_async_copy(rows_fp.at[bn],
                                      out_hbm.at[pos_fp_g], sem_s).wait()

            @pl.when(ai_p > 0)
            def _():
                pltpu.make_async_copy(rows_fb.at[bn],
                                      out_hbm.at[pos_fb_g], sem_s).wait()

            gn = jnp.minimum(g + 1, n_groups - 1)
            fpi_n, fbi_n, pos_fp_n, pos_fb_n, av_n, ai_n = params(gn)

            @pl.when(g + 1 < n_groups)
            def _():
                gathers(fpi_n, fbi_n, bn)

            pltpu.make_async_copy(fpm_hbm.at[fpi_n], rows_fp.at[b],
                                  sem_g).wait()
            pltpu.make_async_copy(fbt_hbm.at[fbi_n], rows_fb.at[b],
                                  sem_g).wait()

            @pl.when(av_g > 0)
            def _():
                pltpu.async_copy(rows_fp.at[b], out_hbm.at[pos_fp_g], sem_s)

            @pl.when(ai_g > 0)
            def _():
                pltpu.async_copy(rows_fb.at[b], out_hbm.at[pos_fb_g], sem_s)

            return pos_fp_n, pos_fb_n, av_n, ai_n, av_g, ai_g

        carry = lax.fori_loop(
            0, n_groups, step,
            (pos_fp0, pos_fb0, av0, ai0, jnp.int32(0), jnp.int32(0)))
        pos_fp_l, pos_fb_l, _, _, av_l, ai_l = carry
        b_l = lax.rem(jnp.int32(n_groups - 1), 2)

        @pl.when(av_l > 0)
        def _():
            pltpu.make_async_copy(rows_fp.at[b_l],
                                  out_hbm.at[pos_fp_l], sem_s).wait()

        @pl.when(ai_l > 0)
        def _():
            pltpu.make_async_copy(rows_fb.at[b_l],
                                  out_hbm.at[pos_fb_l], sem_s).wait()

    return sc_kernel


def kernel(fp_idx, fb_idx, is_valid, fingerprint_matrix, fallback_table):
    B = fp_idx.shape[0]
    D = fingerprint_matrix.shape[1]
    sc = _build(B, fingerprint_matrix.shape[0], fallback_table.shape[0], D)
    return sc(fp_idx.astype(jnp.int32),
              fb_idx.astype(jnp.int32),
              is_valid.astype(jnp.int32),
              fingerprint_matrix,
              fallback_table)
